# baseline (device time: 26041 ns/iter reference)
import jax
import jax.numpy as jnp
from jax import lax
from jax.experimental import pallas as pl
from jax.experimental.pallas import tpu as pltpu

T = 1024
D = 1024
V_SHARD = 8192
NQ = 4
TQ = T // NQ
NC = 4
R = TQ // NC


def _fused(idx, local2d, E):

    def body(idx_ref, loc_ref, e_ref, out_ref,
             rows, part, recv_y, gsems, send_sems, recv_sems):
        xi = lax.axis_index("x")
        yi = lax.axis_index("y")
        zi = lax.axis_index("z")

        x_partner = (1 - xi, yi, zi)
        y_partner = (xi, 1 - yi, zi)
        z_partner = (xi, yi, 1 - zi)

        q_a = 2 * xi + zi
        q_b = 2 * (1 - xi) + zi
        q_c = 2 * xi + (1 - zi)
        a0 = q_a * TQ
        b0 = q_b * TQ
        c0 = q_c * TQ

        barrier_sem = pltpu.get_barrier_semaphore()
        for nbr in (x_partner, y_partner, z_partner):
            pl.semaphore_signal(
                barrier_sem, inc=1,
                device_id=nbr, device_id_type=pl.DeviceIdType.MESH,
            )
        pl.semaphore_wait(barrier_sem, 3)

        rdma_y = {}
        rdma_x = {}
        rdma_z = {}
        rdma_f = {}
        L1, L2, L3, L4 = 1, 2, 3, 4
        for t in range(NC + L4):
            if t < NC:
                def issue_row(i, _, t=t):
                    pltpu.make_async_copy(
                        e_ref.at[pl.ds(idx_ref[i], 1), :],
                        rows.at[pl.ds(i, 1), :],
                        gsems.at[t],
                    ).start()
                    return 0

                lax.fori_loop(t * R, (t + 1) * R, issue_row, 0, unroll=8)

            if 0 <= t - L1 < NC:
                c = t - L1
                pltpu.make_async_copy(
                    e_ref.at[pl.ds(0, R), :],
                    rows.at[pl.ds(c * R, R), :],
                    gsems.at[c],
                ).wait()
                sl = pl.ds(c * R, R)
                loc_c = loc_ref[sl, :]
                mask_c = jnp.logical_and(loc_c >= 0, loc_c < V_SHARD)
                part[sl, :] = jnp.where(
                    mask_c, rows[sl, :], 0.0
                ).astype(jnp.bfloat16)
                r = pltpu.make_async_remote_copy(
                    src_ref=part.at[sl, :],
                    dst_ref=recv_y.at[sl, :],
                    send_sem=send_sems.at[0, c], recv_sem=recv_sems.at[0, c],
                    device_id=y_partner, device_id_type=pl.DeviceIdType.MESH,
                )
                r.start()
                rdma_y[c] = r

            if 0 <= t - L2 < NC:
                c = t - L2
                rdma_y[c].wait()
                sl = pl.ds(c * R, R)
                out_ref[pl.ds(a0 + c * R, R), :] = part[sl, :] + recv_y[sl, :]
                for k, tgt, dct in ((1, x_partner, rdma_x),
                                    (2, z_partner, rdma_z)):
                    r = pltpu.make_async_remote_copy(
                        src_ref=out_ref.at[pl.ds(a0 + c * R, R), :],
                        dst_ref=out_ref.at[pl.ds(a0 + c * R, R), :],
                        send_sem=send_sems.at[k, c], recv_sem=recv_sems.at[k, c],
                        device_id=tgt, device_id_type=pl.DeviceIdType.MESH,
                    )
                    r.start()
                    dct[c] = r

            if 0 <= t - L3 < NC:
                c = t - L3
                rdma_x[c].wait()
                rdma_z[c].wait()
                if c < NC // 2:
                    src0, tgt = b0 + c * R, z_partner
                else:
                    src0, tgt = c0 + c * R, x_partner
                r = pltpu.make_async_remote_copy(
                    src_ref=out_ref.at[pl.ds(src0, R), :],
                    dst_ref=out_ref.at[pl.ds(src0, R), :],
                    send_sem=send_sems.at[3, c], recv_sem=recv_sems.at[3, c],
                    device_id=tgt, device_id_type=pl.DeviceIdType.MESH,
                )
                r.start()
                rdma_f[c] = r

            if 0 <= t - L4 < NC:
                rdma_f[t - L4].wait()

    return pl.pallas_call(
        body,
        out_shape=jax.ShapeDtypeStruct((T, D), jnp.bfloat16),
        in_specs=[
            pl.BlockSpec(memory_space=pltpu.SMEM),
            pl.BlockSpec(memory_space=pltpu.VMEM),
            pl.BlockSpec(memory_space=pl.ANY),
        ],
        out_specs=pl.BlockSpec(memory_space=pltpu.VMEM),
        scratch_shapes=[
            pltpu.VMEM((TQ, D), jnp.float32),
            pltpu.VMEM((TQ, D), jnp.bfloat16),
            pltpu.VMEM((TQ, D), jnp.bfloat16),
            pltpu.SemaphoreType.DMA((NC,)),
            pltpu.SemaphoreType.DMA((4, NC)),
            pltpu.SemaphoreType.DMA((4, NC)),
        ],
        compiler_params=pltpu.CompilerParams(collective_id=0),
    )(idx, local2d, E)


def kernel(ids, E):
    xi = lax.axis_index("x")
    yi = lax.axis_index("y")
    zi = lax.axis_index("z")
    q = 2 * xi + zi

    my_ids = lax.dynamic_slice(ids, (q * TQ,), (TQ,))
    local = my_ids - yi * V_SHARD
    idx = jnp.clip(local, 0, V_SHARD - 1)
    local2d = local[:, None]

    return _fused(idx, local2d, E)


# device time: 24160 ns/iter; 1.0779x vs baseline; 1.0779x over previous
import jax
import jax.numpy as jnp
from jax import lax
from jax.experimental import pallas as pl
from jax.experimental.pallas import tpu as pltpu

T = 1024
D = 1024
V_SHARD = 8192
NQ = 4
TQ = T // NQ
NC = 8
R = TQ // NC


def _fused(idx, local2d, E):

    def body(idx_ref, loc_ref, e_ref, out_ref,
             rows, part, recv_y, gsems, send_sems, recv_sems):
        xi = lax.axis_index("x")
        yi = lax.axis_index("y")
        zi = lax.axis_index("z")

        x_partner = (1 - xi, yi, zi)
        y_partner = (xi, 1 - yi, zi)
        z_partner = (xi, yi, 1 - zi)

        q_a = 2 * xi + zi
        q_b = 2 * (1 - xi) + zi
        q_c = 2 * xi + (1 - zi)
        a0 = q_a * TQ
        b0 = q_b * TQ
        c0 = q_c * TQ

        barrier_sem = pltpu.get_barrier_semaphore()
        for nbr in (x_partner, y_partner, z_partner):
            pl.semaphore_signal(
                barrier_sem, inc=1,
                device_id=nbr, device_id_type=pl.DeviceIdType.MESH,
            )
        pl.semaphore_wait(barrier_sem, 3)

        rdma_y = {}
        rdma_x = {}
        rdma_z = {}
        rdma_f = {}
        L1, L2, L3, L4 = 1, 3, 5, 7
        for t in range(NC + L4):
            if t < NC:
                def issue_row(i, _, t=t):
                    pltpu.make_async_copy(
                        e_ref.at[pl.ds(idx_ref[i], 1), :],
                        rows.at[pl.ds(i, 1), :],
                        gsems.at[t],
                    ).start()
                    return 0

                lax.fori_loop(t * R, (t + 1) * R, issue_row, 0, unroll=8)

            if 0 <= t - L1 < NC:
                c = t - L1
                pltpu.make_async_copy(
                    e_ref.at[pl.ds(0, R), :],
                    rows.at[pl.ds(c * R, R), :],
                    gsems.at[c],
                ).wait()
                sl = pl.ds(c * R, R)
                loc_c = loc_ref[sl, :]
                mask_c = jnp.logical_and(loc_c >= 0, loc_c < V_SHARD)
                part[sl, :] = jnp.where(
                    mask_c, rows[sl, :], 0.0
                ).astype(jnp.bfloat16)
                r = pltpu.make_async_remote_copy(
                    src_ref=part.at[sl, :],
                    dst_ref=recv_y.at[sl, :],
                    send_sem=send_sems.at[0, c], recv_sem=recv_sems.at[0, c],
                    device_id=y_partner, device_id_type=pl.DeviceIdType.MESH,
                )
                r.start()
                rdma_y[c] = r

            if 0 <= t - L2 < NC:
                c = t - L2
                rdma_y[c].wait()
                sl = pl.ds(c * R, R)
                out_ref[pl.ds(a0 + c * R, R), :] = part[sl, :] + recv_y[sl, :]
                for k, tgt, dct in ((1, x_partner, rdma_x),
                                    (2, z_partner, rdma_z)):
                    r = pltpu.make_async_remote_copy(
                        src_ref=out_ref.at[pl.ds(a0 + c * R, R), :],
                        dst_ref=out_ref.at[pl.ds(a0 + c * R, R), :],
                        send_sem=send_sems.at[k, c], recv_sem=recv_sems.at[k, c],
                        device_id=tgt, device_id_type=pl.DeviceIdType.MESH,
                    )
                    r.start()
                    dct[c] = r

            if 0 <= t - L3 < NC:
                c = t - L3
                if c < NC // 2:
                    rdma_x[c].wait()
                    src0, tgt = b0 + c * R, z_partner
                else:
                    rdma_z[c].wait()
                    src0, tgt = c0 + c * R, x_partner
                r = pltpu.make_async_remote_copy(
                    src_ref=out_ref.at[pl.ds(src0, R), :],
                    dst_ref=out_ref.at[pl.ds(src0, R), :],
                    send_sem=send_sems.at[3, c], recv_sem=recv_sems.at[3, c],
                    device_id=tgt, device_id_type=pl.DeviceIdType.MESH,
                )
                r.start()
                rdma_f[c] = r

            if 0 <= t - L4 < NC:
                rdma_f[t - L4].wait()

        for c in range(NC):
            if c < NC // 2:
                rdma_z[c].wait()
            else:
                rdma_x[c].wait()

    return pl.pallas_call(
        body,
        out_shape=jax.ShapeDtypeStruct((T, D), jnp.bfloat16),
        in_specs=[
            pl.BlockSpec(memory_space=pltpu.SMEM),
            pl.BlockSpec(memory_space=pltpu.VMEM),
            pl.BlockSpec(memory_space=pl.ANY),
        ],
        out_specs=pl.BlockSpec(memory_space=pltpu.VMEM),
        scratch_shapes=[
            pltpu.VMEM((TQ, D), jnp.float32),
            pltpu.VMEM((TQ, D), jnp.bfloat16),
            pltpu.VMEM((TQ, D), jnp.bfloat16),
            pltpu.SemaphoreType.DMA((NC,)),
            pltpu.SemaphoreType.DMA((4, NC)),
            pltpu.SemaphoreType.DMA((4, NC)),
        ],
        compiler_params=pltpu.CompilerParams(collective_id=0),
    )(idx, local2d, E)


def kernel(ids, E):
    xi = lax.axis_index("x")
    yi = lax.axis_index("y")
    zi = lax.axis_index("z")
    q = 2 * xi + zi

    my_ids = lax.dynamic_slice(ids, (q * TQ,), (TQ,))
    local = my_ids - yi * V_SHARD
    idx = jnp.clip(local, 0, V_SHARD - 1)
    local2d = local[:, None]

    return _fused(idx, local2d, E)
